# baseline (device time: 102439 ns/iter reference)
import jax
import jax.numpy as jnp
from jax import lax
from jax.experimental import pallas as pl
from jax.experimental.pallas import tpu as pltpu

N_DEV = 16
K_SUB = 4

PERM = [0, 1, 5, 4, 8, 9, 13, 12, 15, 14, 10, 11, 7, 6, 2, 3]
INVPERM = [PERM.index(i) for i in range(N_DEV)]


def kernel(x, w_mat):
    m, k = x.shape
    _, n = w_mat.shape
    m_per = m // N_DEV
    nh = n // 2
    m_sub = m_per // K_SUB

    p = lax.axis_index("i")
    perm = jnp.asarray(PERM, jnp.int32)
    q = jnp.asarray(INVPERM, jnp.int32)[p]
    s_arr = jnp.arange(N_DEV - 1, dtype=jnp.int32)
    right = perm[jnp.mod(q + 1, N_DEV)]
    left = perm[jnp.mod(q - 1, N_DEV)]
    fcs = perm[jnp.mod(q - 1 - s_arr, N_DEV)]
    bcs = perm[jnp.mod(q + 1 + s_arr, N_DEV)]
    idx = jnp.concatenate(
        [right[None], left[None], fcs, bcs]
    ).astype(jnp.int32)

    def body(idx_ref, x_ref, w_ref, out_ref,
             fsend_ref, bsend_ref, frecv_ref, brecv_ref,
             fsend_sems, bsend_sems, frecv_sems, brecv_sems):
        my = lax.axis_index("i")
        right = idx_ref[0]
        left = idx_ref[1]

        barrier_sem = pltpu.get_barrier_semaphore()
        for nbr in (left, right):
            pl.semaphore_signal(
                barrier_sem, inc=1,
                device_id=(nbr,), device_id_type=pl.DeviceIdType.MESH,
            )
        pl.semaphore_wait(barrier_sem, 2)

        def sub_dot(c, ks, lo):
            xs = x_ref[pl.ds(c * m_per + ks * m_sub, m_sub), :]
            return jax.lax.dot_general(
                xs, w_ref[:, pl.ds(lo, nh)],
                (((1,), (0,)), ((), ())),
                preferred_element_type=jnp.float32,
            )

        def make_rdma(s, ks, fwd):
            slot = s % 2
            send_ref = fsend_ref if fwd else bsend_ref
            recv_ref = frecv_ref if fwd else brecv_ref
            send_sems = fsend_sems if fwd else bsend_sems
            recv_sems = frecv_sems if fwd else brecv_sems
            return pltpu.make_async_remote_copy(
                src_ref=send_ref.at[slot, pl.ds(ks * m_sub, m_sub), :],
                dst_ref=recv_ref.at[s, pl.ds(ks * m_sub, m_sub), :],
                send_sem=send_sems.at[s, ks],
                recv_sem=recv_sems.at[s, ks],
                device_id=(right if fwd else left,),
                device_id_type=pl.DeviceIdType.MESH,
            )

        rows = lambda ks: pl.ds(ks * m_sub, m_sub)

        for ks in range(K_SUB):
            fsend_ref[0, rows(ks), :] = sub_dot(
                idx_ref[2], ks, 0
            ).astype(jnp.bfloat16)
            make_rdma(0, ks, True).start()
            bsend_ref[0, rows(ks), :] = sub_dot(
                idx_ref[2 + (N_DEV - 1)], ks, nh
            ).astype(jnp.bfloat16)
            make_rdma(0, ks, False).start()

        for s in range(1, N_DEV - 1):
            slot = s % 2
            fc = idx_ref[2 + s]
            bc = idx_ref[2 + (N_DEV - 1) + s]
            for ks in range(K_SUB):
                fdot = sub_dot(fc, ks, 0)
                bdot = sub_dot(bc, ks, nh)
                if s >= 2:
                    make_rdma(s - 2, ks, True).wait_send()
                    make_rdma(s - 2, ks, False).wait_send()
                make_rdma(s - 1, ks, True).wait_recv()
                fsend_ref[slot, rows(ks), :] = (
                    frecv_ref[s - 1, rows(ks), :].astype(jnp.float32) + fdot
                ).astype(jnp.bfloat16)
                make_rdma(s, ks, True).start()

                make_rdma(s - 1, ks, False).wait_recv()
                bsend_ref[slot, rows(ks), :] = (
                    brecv_ref[s - 1, rows(ks), :].astype(jnp.float32) + bdot
                ).astype(jnp.bfloat16)
                make_rdma(s, ks, False).start()

        for ks in range(K_SUB):
            fdot = sub_dot(my, ks, 0)
            bdot = sub_dot(my, ks, nh)
            make_rdma(N_DEV - 2, ks, True).wait_recv()
            out_ref[rows(ks), pl.ds(0, nh)] = (
                frecv_ref[N_DEV - 2, rows(ks), :].astype(jnp.float32) + fdot
            )
            make_rdma(N_DEV - 2, ks, False).wait_recv()
            out_ref[rows(ks), pl.ds(nh, nh)] = (
                brecv_ref[N_DEV - 2, rows(ks), :].astype(jnp.float32) + bdot
            )

        for s in (N_DEV - 3, N_DEV - 2):
            for ks in range(K_SUB):
                make_rdma(s, ks, True).wait_send()
                make_rdma(s, ks, False).wait_send()

    return pl.pallas_call(
        body,
        out_shape=jax.ShapeDtypeStruct((m_per, n), jnp.float32),
        in_specs=[
            pl.BlockSpec(memory_space=pltpu.SMEM),
            pl.BlockSpec(memory_space=pltpu.VMEM),
            pl.BlockSpec(memory_space=pltpu.VMEM),
        ],
        out_specs=pl.BlockSpec(memory_space=pltpu.VMEM),
        scratch_shapes=[
            pltpu.VMEM((2, m_per, nh), jnp.bfloat16),
            pltpu.VMEM((2, m_per, nh), jnp.bfloat16),
            pltpu.VMEM((N_DEV - 1, m_per, nh), jnp.bfloat16),
            pltpu.VMEM((N_DEV - 1, m_per, nh), jnp.bfloat16),
            pltpu.SemaphoreType.DMA((N_DEV - 1, K_SUB)),
            pltpu.SemaphoreType.DMA((N_DEV - 1, K_SUB)),
            pltpu.SemaphoreType.DMA((N_DEV - 1, K_SUB)),
            pltpu.SemaphoreType.DMA((N_DEV - 1, K_SUB)),
        ],
        compiler_params=pltpu.CompilerParams(collective_id=0),
    )(idx, x, w_mat)


# device time: 99385 ns/iter; 1.0307x vs baseline; 1.0307x over previous
import jax
import jax.numpy as jnp
from jax import lax
from jax.experimental import pallas as pl
from jax.experimental.pallas import tpu as pltpu

N_DEV = 16
K_SUB = 4


def kernel(x, w_mat):
    m, k = x.shape
    _, n = w_mat.shape
    m_per = m // N_DEV
    nh = n // 2
    m_sub = m_per // K_SUB

    def body(x_ref, w_ref, out_ref,
             fsend_ref, bsend_ref, frecv_ref, brecv_ref,
             fsend_sems, bsend_sems, frecv_sems, brecv_sems):
        p = lax.axis_index("i")
        right = lax.rem(p + 1, N_DEV)
        left = lax.rem(p - 1 + N_DEV, N_DEV)

        barrier_sem = pltpu.get_barrier_semaphore()
        for nbr in (left, right):
            pl.semaphore_signal(
                barrier_sem, inc=1,
                device_id=(nbr,), device_id_type=pl.DeviceIdType.MESH,
            )
        pl.semaphore_wait(barrier_sem, 2)

        def sub_dot(c, ks, lo):
            xs = x_ref[pl.ds(c * m_per + ks * m_sub, m_sub), :]
            return jax.lax.dot_general(
                xs, w_ref[:, pl.ds(lo, nh)],
                (((1,), (0,)), ((), ())),
                preferred_element_type=jnp.float32,
            )

        def make_rdma(s, ks, fwd):
            slot = s % 2
            send_ref = fsend_ref if fwd else bsend_ref
            recv_ref = frecv_ref if fwd else brecv_ref
            send_sems = fsend_sems if fwd else bsend_sems
            recv_sems = frecv_sems if fwd else brecv_sems
            return pltpu.make_async_remote_copy(
                src_ref=send_ref.at[slot, pl.ds(ks * m_sub, m_sub), :],
                dst_ref=recv_ref.at[s, pl.ds(ks * m_sub, m_sub), :],
                send_sem=send_sems.at[s, ks],
                recv_sem=recv_sems.at[s, ks],
                device_id=(right if fwd else left,),
                device_id_type=pl.DeviceIdType.MESH,
            )

        rows = lambda ks: pl.ds(ks * m_sub, m_sub)

        fc0 = lax.rem(p - 1 + N_DEV, N_DEV)
        bc0 = lax.rem(p + 1, N_DEV)
        for ks in range(K_SUB):
            fsend_ref[0, rows(ks), :] = sub_dot(fc0, ks, 0).astype(jnp.bfloat16)
            make_rdma(0, ks, True).start()
            bsend_ref[0, rows(ks), :] = sub_dot(bc0, ks, nh).astype(jnp.bfloat16)
            make_rdma(0, ks, False).start()

        fc1 = lax.rem(p - 2 + N_DEV, N_DEV)
        bc1 = lax.rem(p + 2, N_DEV)
        fdots = [sub_dot(fc1, ks, 0) for ks in range(K_SUB)]
        bdots = [sub_dot(bc1, ks, nh) for ks in range(K_SUB)]

        for s in range(1, N_DEV - 1):
            slot = s % 2
            nfc = lax.rem(p - s - 2 + 2 * N_DEV, N_DEV) if s < N_DEV - 2 else p
            nbc = lax.rem(p + s + 2, N_DEV) if s < N_DEV - 2 else p
            nfdots = [sub_dot(nfc, ks, 0) for ks in range(K_SUB)]
            nbdots = [sub_dot(nbc, ks, nh) for ks in range(K_SUB)]
            for ks in range(K_SUB):
                if s >= 2:
                    make_rdma(s - 2, ks, True).wait_send()
                    make_rdma(s - 2, ks, False).wait_send()
                make_rdma(s - 1, ks, True).wait_recv()
                fsend_ref[slot, rows(ks), :] = (
                    frecv_ref[s - 1, rows(ks), :].astype(jnp.float32) + fdots[ks]
                ).astype(jnp.bfloat16)
                make_rdma(s, ks, True).start()

                make_rdma(s - 1, ks, False).wait_recv()
                bsend_ref[slot, rows(ks), :] = (
                    brecv_ref[s - 1, rows(ks), :].astype(jnp.float32) + bdots[ks]
                ).astype(jnp.bfloat16)
                make_rdma(s, ks, False).start()
            fdots, bdots = nfdots, nbdots

        for ks in range(K_SUB):
            make_rdma(N_DEV - 2, ks, True).wait_recv()
            out_ref[rows(ks), pl.ds(0, nh)] = (
                frecv_ref[N_DEV - 2, rows(ks), :].astype(jnp.float32) + fdots[ks]
            )
            make_rdma(N_DEV - 2, ks, False).wait_recv()
            out_ref[rows(ks), pl.ds(nh, nh)] = (
                brecv_ref[N_DEV - 2, rows(ks), :].astype(jnp.float32) + bdots[ks]
            )

        for s in (N_DEV - 3, N_DEV - 2):
            for ks in range(K_SUB):
                make_rdma(s, ks, True).wait_send()
                make_rdma(s, ks, False).wait_send()

    return pl.pallas_call(
        body,
        out_shape=jax.ShapeDtypeStruct((m_per, n), jnp.float32),
        in_specs=[
            pl.BlockSpec(memory_space=pltpu.VMEM),
            pl.BlockSpec(memory_space=pltpu.VMEM),
        ],
        out_specs=pl.BlockSpec(memory_space=pltpu.VMEM),
        scratch_shapes=[
            pltpu.VMEM((2, m_per, nh), jnp.bfloat16),
            pltpu.VMEM((2, m_per, nh), jnp.bfloat16),
            pltpu.VMEM((N_DEV - 1, m_per, nh), jnp.bfloat16),
            pltpu.VMEM((N_DEV - 1, m_per, nh), jnp.bfloat16),
            pltpu.SemaphoreType.DMA((N_DEV - 1, K_SUB)),
            pltpu.SemaphoreType.DMA((N_DEV - 1, K_SUB)),
            pltpu.SemaphoreType.DMA((N_DEV - 1, K_SUB)),
            pltpu.SemaphoreType.DMA((N_DEV - 1, K_SUB)),
        ],
        compiler_params=pltpu.CompilerParams(collective_id=0),
    )(x, w_mat)


# device time: 98381 ns/iter; 1.0412x vs baseline; 1.0102x over previous
import jax
import jax.numpy as jnp
from jax import lax
from jax.experimental import pallas as pl
from jax.experimental.pallas import tpu as pltpu

N_DEV = 16
K_SUB = 2


def kernel(x, w_mat):
    m, k = x.shape
    _, n = w_mat.shape
    m_per = m // N_DEV
    nh = n // 2
    m_sub = m_per // K_SUB

    def body(x_ref, w_ref, out_ref,
             fsend_ref, bsend_ref, frecv_ref, brecv_ref,
             fsend_sems, bsend_sems, frecv_sems, brecv_sems):
        p = lax.axis_index("i")
        right = lax.rem(p + 1, N_DEV)
        left = lax.rem(p - 1 + N_DEV, N_DEV)

        barrier_sem = pltpu.get_barrier_semaphore()
        for nbr in (left, right):
            pl.semaphore_signal(
                barrier_sem, inc=1,
                device_id=(nbr,), device_id_type=pl.DeviceIdType.MESH,
            )
        pl.semaphore_wait(barrier_sem, 2)

        def sub_dot(c, ks, lo):
            xs = x_ref[pl.ds(c * m_per + ks * m_sub, m_sub), :]
            return jax.lax.dot_general(
                xs, w_ref[:, pl.ds(lo, nh)],
                (((1,), (0,)), ((), ())),
                preferred_element_type=jnp.float32,
            )

        def make_rdma(s, ks, fwd):
            slot = s % 2
            send_ref = fsend_ref if fwd else bsend_ref
            recv_ref = frecv_ref if fwd else brecv_ref
            send_sems = fsend_sems if fwd else bsend_sems
            recv_sems = frecv_sems if fwd else brecv_sems
            return pltpu.make_async_remote_copy(
                src_ref=send_ref.at[slot, pl.ds(ks * m_sub, m_sub), :],
                dst_ref=recv_ref.at[s, pl.ds(ks * m_sub, m_sub), :],
                send_sem=send_sems.at[s, ks],
                recv_sem=recv_sems.at[s, ks],
                device_id=(right if fwd else left,),
                device_id_type=pl.DeviceIdType.MESH,
            )

        rows = lambda ks: pl.ds(ks * m_sub, m_sub)

        fc0 = lax.rem(p - 1 + N_DEV, N_DEV)
        bc0 = lax.rem(p + 1, N_DEV)
        for ks in range(K_SUB):
            fsend_ref[0, rows(ks), :] = sub_dot(fc0, ks, 0).astype(jnp.bfloat16)
            make_rdma(0, ks, True).start()
            bsend_ref[0, rows(ks), :] = sub_dot(bc0, ks, nh).astype(jnp.bfloat16)
            make_rdma(0, ks, False).start()

        fc1 = lax.rem(p - 2 + N_DEV, N_DEV)
        bc1 = lax.rem(p + 2, N_DEV)
        fdots = [sub_dot(fc1, ks, 0) for ks in range(K_SUB)]
        bdots = [sub_dot(bc1, ks, nh) for ks in range(K_SUB)]

        for s in range(1, N_DEV - 1):
            slot = s % 2
            nfc = lax.rem(p - s - 2 + 2 * N_DEV, N_DEV) if s < N_DEV - 2 else p
            nbc = lax.rem(p + s + 2, N_DEV) if s < N_DEV - 2 else p
            nfdots = [sub_dot(nfc, ks, 0) for ks in range(K_SUB)]
            nbdots = [sub_dot(nbc, ks, nh) for ks in range(K_SUB)]
            for ks in range(K_SUB):
                if s >= 2:
                    make_rdma(s - 2, ks, True).wait_send()
                    make_rdma(s - 2, ks, False).wait_send()
                make_rdma(s - 1, ks, True).wait_recv()
                fsend_ref[slot, rows(ks), :] = (
                    frecv_ref[s - 1, rows(ks), :].astype(jnp.float32) + fdots[ks]
                ).astype(jnp.bfloat16)
                make_rdma(s, ks, True).start()

                make_rdma(s - 1, ks, False).wait_recv()
                bsend_ref[slot, rows(ks), :] = (
                    brecv_ref[s - 1, rows(ks), :].astype(jnp.float32) + bdots[ks]
                ).astype(jnp.bfloat16)
                make_rdma(s, ks, False).start()
            fdots, bdots = nfdots, nbdots

        for ks in range(K_SUB):
            make_rdma(N_DEV - 2, ks, True).wait_recv()
            out_ref[rows(ks), pl.ds(0, nh)] = (
                frecv_ref[N_DEV - 2, rows(ks), :].astype(jnp.float32) + fdots[ks]
            )
            make_rdma(N_DEV - 2, ks, False).wait_recv()
            out_ref[rows(ks), pl.ds(nh, nh)] = (
                brecv_ref[N_DEV - 2, rows(ks), :].astype(jnp.float32) + bdots[ks]
            )

        for s in (N_DEV - 3, N_DEV - 2):
            for ks in range(K_SUB):
                make_rdma(s, ks, True).wait_send()
                make_rdma(s, ks, False).wait_send()

    return pl.pallas_call(
        body,
        out_shape=jax.ShapeDtypeStruct((m_per, n), jnp.float32),
        in_specs=[
            pl.BlockSpec(memory_space=pltpu.VMEM),
            pl.BlockSpec(memory_space=pltpu.VMEM),
        ],
        out_specs=pl.BlockSpec(memory_space=pltpu.VMEM),
        scratch_shapes=[
            pltpu.VMEM((2, m_per, nh), jnp.bfloat16),
            pltpu.VMEM((2, m_per, nh), jnp.bfloat16),
            pltpu.VMEM((N_DEV - 1, m_per, nh), jnp.bfloat16),
            pltpu.VMEM((N_DEV - 1, m_per, nh), jnp.bfloat16),
            pltpu.SemaphoreType.DMA((N_DEV - 1, K_SUB)),
            pltpu.SemaphoreType.DMA((N_DEV - 1, K_SUB)),
            pltpu.SemaphoreType.DMA((N_DEV - 1, K_SUB)),
            pltpu.SemaphoreType.DMA((N_DEV - 1, K_SUB)),
        ],
        compiler_params=pltpu.CompilerParams(collective_id=0),
    )(x, w_mat)
